# Initial kernel scaffold; baseline (speedup 1.0000x reference)
import jax
import jax.numpy as jnp
from jax.experimental import pallas as pl


def kernel(features, W, gamma, beta, neighbors):
    raise NotImplementedError("write your pallas kernel here")


# speedup vs baseline: 1.6467x; 1.6467x over previous
import functools

import jax
import jax.numpy as jnp
from jax import lax
from jax.experimental import pallas as pl
from jax.experimental.pallas import tpu as pltpu
from jax.experimental.pallas import tpu_sc as plsc

F32 = jnp.float32
I32 = jnp.int32

_CHUNK = 128
_BT = 512
_NC, _NS = 2, 16
_NW = _NC * _NS
_LANES = 16


def _xform_body(f_ref, w_ref, t_ref):
    f = f_ref[...]
    for k in range(w_ref.shape[0]):
        t_ref[k] = jnp.dot(f, w_ref[k], preferred_element_type=F32)


@functools.lru_cache(maxsize=None)
def _make_xform(npad, kk, c_in, c_out):
    return pl.pallas_call(
        _xform_body,
        grid=(npad // _BT,),
        in_specs=[
            pl.BlockSpec((_BT, c_in), lambda i: (i, 0)),
            pl.BlockSpec((kk, c_in, c_out), lambda i: (0, 0, 0)),
        ],
        out_specs=pl.BlockSpec((kk, _BT, c_out), lambda i: (0, i, 0)),
        out_shape=jax.ShapeDtypeStruct((kk, npad, c_out), F32),
    )


@functools.lru_cache(maxsize=None)
def _make_sc(npad, kk, c_out, n_real):
    nchunks = npad // _CHUNK
    nt = -(-nchunks // _NW)
    center = kk // 2
    mesh = plsc.VectorSubcoreMesh(
        core_axis_name="c", subcore_axis_name="s",
        num_cores=_NC, num_subcores=_NS)

    def body(neigh_hbm, t_hbm, out_hbm, idx_v, acc_v, sem):
        wid = lax.axis_index("s") * _NC + lax.axis_index("c")

        def do_chunk(c):
            base = c * _CHUNK
            pltpu.sync_copy(neigh_hbm.at[:, pl.ds(base, _CHUNK)], idx_v)
            init = pltpu.async_copy(
                t_hbm.at[pl.ds(center * npad + base, _CHUNK)], acc_v, sem)
            for k in range(kk):
                if k == center:
                    continue

                def trans(i, _, k=k):
                    v = idx_v[k, pl.ds(i * _LANES, _LANES)]
                    g = jnp.where(v < 0, n_real, v) + (k * npad)
                    idx_v[k, pl.ds(i * _LANES, _LANES)] = g
                    return 0

                lax.fori_loop(0, _CHUNK // _LANES, trans, 0)
            init.wait()
            descs = [
                pltpu.async_copy(t_hbm.at[idx_v.at[k]], acc_v, sem, add=True)
                for k in range(kk) if k != center
            ]
            for d in descs:
                d.wait()
            pltpu.sync_copy(acc_v, out_hbm.at[pl.ds(base, _CHUNK)])

        def tbody(t, carry):
            c = t * _NW + wid

            @pl.when(c < nchunks)
            def _():
                do_chunk(c)

            return carry

        lax.fori_loop(0, nt, tbody, 0)

    return pl.kernel(
        body,
        out_type=jax.ShapeDtypeStruct((npad, c_out), F32),
        mesh=mesh,
        scratch_types=[
            pltpu.VMEM((kk, _CHUNK), I32),
            pltpu.VMEM((_CHUNK, c_out), F32),
            pltpu.SemaphoreType.DMA,
        ],
    )


def _stats_body(x_ref, s_ref, q_ref, accs, accq):
    i = pl.program_id(0)
    x = x_ref[...]

    @pl.when(i == 0)
    def _():
        accs[...] = jnp.zeros_like(accs)
        accq[...] = jnp.zeros_like(accq)

    accs[...] += jnp.sum(x, axis=0, keepdims=True)
    accq[...] += jnp.sum(x * x, axis=0, keepdims=True)

    @pl.when(i == pl.num_programs(0) - 1)
    def _():
        s_ref[...] = accs[...]
        q_ref[...] = accq[...]


@functools.lru_cache(maxsize=None)
def _make_stats(npad, c_out):
    return pl.pallas_call(
        _stats_body,
        grid=(npad // _BT,),
        in_specs=[pl.BlockSpec((_BT, c_out), lambda i: (i, 0))],
        out_specs=[
            pl.BlockSpec((1, c_out), lambda i: (0, 0)),
            pl.BlockSpec((1, c_out), lambda i: (0, 0)),
        ],
        out_shape=[
            jax.ShapeDtypeStruct((1, c_out), F32),
            jax.ShapeDtypeStruct((1, c_out), F32),
        ],
        scratch_shapes=[
            pltpu.VMEM((1, c_out), F32),
            pltpu.VMEM((1, c_out), F32),
        ],
    )


def _norm_body(n_real, x_ref, s_ref, q_ref, g_ref, b_ref, o_ref):
    inv_n = F32(1.0 / n_real)
    mean = s_ref[...] * inv_n
    var = q_ref[...] * inv_n - mean * mean
    rstd = lax.rsqrt(var + F32(1e-4))
    y = (x_ref[...] - mean) * (rstd * g_ref[...]) + b_ref[...]
    o_ref[...] = jnp.maximum(y, F32(0.0))


@functools.lru_cache(maxsize=None)
def _make_norm(npad, c_out, n_real):
    return pl.pallas_call(
        functools.partial(_norm_body, n_real),
        grid=(npad // _BT,),
        in_specs=[
            pl.BlockSpec((_BT, c_out), lambda i: (i, 0)),
            pl.BlockSpec((1, c_out), lambda i: (0, 0)),
            pl.BlockSpec((1, c_out), lambda i: (0, 0)),
            pl.BlockSpec((1, c_out), lambda i: (0, 0)),
            pl.BlockSpec((1, c_out), lambda i: (0, 0)),
        ],
        out_specs=pl.BlockSpec((_BT, c_out), lambda i: (i, 0)),
        out_shape=jax.ShapeDtypeStruct((npad, c_out), F32),
    )


def kernel(features, W, gamma, beta, neighbors):
    n, c_in = features.shape
    kk, _, c_out = W.shape
    npad = (n // _BT + 1) * _BT

    f = jnp.pad(features.astype(F32), ((0, npad - n), (0, 0)))
    neigh_t = jnp.pad(neighbors.astype(I32).T, ((0, 0), (0, npad - n)),
                      constant_values=-1)

    t = _make_xform(npad, kk, c_in, c_out)(f, W.astype(F32))
    t_flat = t.reshape(kk * npad, c_out)
    out_raw = _make_sc(npad, kk, c_out, n)(neigh_t, t_flat)
    s, q = _make_stats(npad, c_out)(out_raw)
    y = _make_norm(npad, c_out, n)(
        out_raw, s, q,
        gamma.reshape(1, c_out).astype(F32),
        beta.reshape(1, c_out).astype(F32))
    return y[:n]
